# baseline (speedup 1.0000x reference)
import jax
import jax.numpy as jnp
from jax.experimental import pallas as pl
from jax.experimental.pallas import tpu as pltpu

_FILL = -10000.0
_V = 100000
_T = 2048
_H = 2048
_R = 65
_RP = 128
_VB = 2048
_NVR = _V // _VB
_TAIL = _V - _NVR * _VB
_TBASE = _NVR * _VB
_RC = 1024
_NRC = _T // _RC
_NSEM = 16
_NPRI = 2


def _overlay(rest, tok_col, base, width):
    cols = jax.lax.broadcasted_iota(jnp.int32, (_RP, width), 1) + base
    ohb = tok_col == cols
    mm = jnp.dot(rest, ohb.astype(jnp.float32), preferred_element_type=jnp.float32)
    return jnp.where(jnp.any(ohb, axis=0)[None, :], mm, _FILL)


def _body(tok_ref, hid_ref, wt_ref, out_ref, fill_ref, ovl_ref, tail_ref,
          rest_ref, sems, ovl_sem, tail_sem):
    fill_ref[...] = jnp.full((_RC, _VB), _FILL, jnp.float32)
    rest_ref[...] = jnp.dot(
        hid_ref[...], wt_ref[...], preferred_element_type=jnp.float32
    )
    toks = tok_ref[...]

    has = []
    for v in range(_NVR):
        base = v * _VB
        has.append(jnp.any((toks >= base) & (toks < base + _VB)))
    has_tail = jnp.any(toks >= _TBASE)

    @pl.when(has_tail)
    def _():
        tail_ref[...] = _overlay(rest_ref[...], tok_ref[:, 0:1], _TBASE, _TAIL)

    @pl.when(jnp.logical_not(has_tail))
    def _():
        tail_ref[...] = jnp.full((_T, _TAIL), _FILL, jnp.float32)

    tail_cp = pltpu.make_async_copy(
        tail_ref, out_ref.at[:, pl.ds(_TBASE, _TAIL)], tail_sem
    )
    tail_cp.start(priority=1)

    ring = []
    for v in range(_NVR):
        for c in range(_NRC):
            cp = pltpu.make_async_copy(
                fill_ref,
                out_ref.at[pl.ds(c * _RC, _RC), pl.ds(v * _VB, _VB)],
                sems.at[len(ring) % _NSEM],
            )
            cond = jnp.logical_not(has[v])
            if len(ring) >= _NSEM:
                prev_cp, prev_cond = ring[len(ring) - _NSEM]
                @pl.when(prev_cond)
                def _(prev_cp=prev_cp):
                    prev_cp.wait()
                ring[len(ring) - _NSEM] = (None, None)
            @pl.when(cond)
            def _(cp=cp, pri=len(ring) % _NPRI):
                cp.start(priority=pri)
            ring.append((cp, cond))

    for v in range(_NVR):
        @pl.when(has[v])
        def _(base=v * _VB):
            ovl_ref[...] = _overlay(rest_ref[...], tok_ref[:, 0:1], base, _VB)
            cp = pltpu.make_async_copy(
                ovl_ref, out_ref.at[:, pl.ds(base, _VB)], ovl_sem
            )
            cp.start()
            cp.wait()

    for cp, cond in ring:
        if cp is None:
            continue
        @pl.when(cond)
        def _(cp=cp):
            cp.wait()
    tail_cp.wait()


def kernel(hidden_states, W, token_ids):
    hid = hidden_states.reshape(_T, _H)
    wt = jnp.zeros((_H, _RP), jnp.float32).at[:, :_R].set(W.T)
    tok = jnp.broadcast_to(
        jnp.full((_RP,), -1, jnp.int32).at[:_R].set(token_ids)[:, None],
        (_RP, 128),
    )
    out = pl.pallas_call(
        _body,
        in_specs=[
            pl.BlockSpec(memory_space=pltpu.MemorySpace.VMEM),
            pl.BlockSpec(memory_space=pltpu.MemorySpace.VMEM),
            pl.BlockSpec(memory_space=pltpu.MemorySpace.VMEM),
        ],
        out_specs=pl.BlockSpec(memory_space=pl.ANY),
        out_shape=jax.ShapeDtypeStruct((_T, _V), jnp.float32),
        scratch_shapes=[
            pltpu.VMEM((_RC, _VB), jnp.float32),
            pltpu.VMEM((_T, _VB), jnp.float32),
            pltpu.VMEM((_T, _TAIL), jnp.float32),
            pltpu.VMEM((_T, _RP), jnp.float32),
            pltpu.SemaphoreType.DMA((_NSEM,)),
            pltpu.SemaphoreType.DMA,
            pltpu.SemaphoreType.DMA,
        ],
        compiler_params=pltpu.CompilerParams(
            vmem_limit_bytes=63 * 1024 * 1024,
        ),
    )(tok, hid, wt)
    return out.reshape(1, _T, _V)


# speedup vs baseline: 1.3313x; 1.3313x over previous
import jax
import jax.numpy as jnp
from jax.experimental import pallas as pl
from jax.experimental.pallas import tpu as pltpu

_FILL = -10000.0
_V = 100000
_T = 2048
_H = 2048
_R = 65
_RP = 128
_RC = 32
_NS = _T // _RC
_CB = 2048
_NCB = (_V + _CB - 1) // _CB


def _body(tok_ref, hid_ref, wt_ref, out_ref, rest_ref):
    v = pl.program_id(0)

    @pl.when(v == 0)
    def _():
        rest_ref[...] = jnp.dot(
            hid_ref[...], wt_ref[...], preferred_element_type=jnp.float32
        )

    toks = tok_ref[...]
    rest = rest_ref[pl.ds(v * _RC, _RC), :]

    for b in range(_NCB):
        base = b * _CB
        width = min(_CB, _V - base)
        has = jnp.any((toks >= base) & (toks < base + width))

        @pl.when(has)
        def _(base=base, width=width):
            cols = jax.lax.broadcasted_iota(jnp.int32, (_RP, width), 1) + base
            ohb = tok_ref[:, 0:1] == cols
            mm = jnp.dot(
                rest, ohb.astype(jnp.float32),
                preferred_element_type=jnp.float32,
            )
            out_ref[:, pl.ds(base, width)] = jnp.where(
                jnp.any(ohb, axis=0)[None, :], mm, _FILL
            )

        @pl.when(jnp.logical_not(has) & (v < 2))
        def _(base=base, width=width):
            out_ref[:, pl.ds(base, width)] = jnp.full(
                (_RC, width), _FILL, jnp.float32
            )


def kernel(hidden_states, W, token_ids):
    hid = hidden_states.reshape(_T, _H)
    wt = jnp.zeros((_H, _RP), jnp.float32).at[:, :_R].set(W.T)
    tok = jnp.broadcast_to(
        jnp.full((_RP,), -1, jnp.int32).at[:_R].set(token_ids)[:, None],
        (_RP, 128),
    )
    out = pl.pallas_call(
        _body,
        grid=(_NS,),
        in_specs=[
            pl.BlockSpec((_RP, 128), lambda v: (0, 0)),
            pl.BlockSpec((_T, _H), lambda v: (0, 0)),
            pl.BlockSpec((_H, _RP), lambda v: (0, 0)),
        ],
        out_specs=pl.BlockSpec((_RC, _V), lambda v: (v, 0)),
        out_shape=jax.ShapeDtypeStruct((_T, _V), jnp.float32),
        scratch_shapes=[pltpu.VMEM((_T, _RP), jnp.float32)],
        compiler_params=pltpu.CompilerParams(
            dimension_semantics=("arbitrary",),
            vmem_limit_bytes=63 * 1024 * 1024,
        ),
    )(tok, hid, wt)
    return out.reshape(1, _T, _V)
